# baseline (speedup 1.0000x reference)
import functools

import jax
import jax.numpy as jnp
from jax import lax
from jax.experimental import pallas as pl
from jax.experimental.pallas import tpu as pltpu
from jax.experimental.pallas import tpu_sc as plsc

_B = 1024
_N = 100000
_L = 16
_NW = 32
_BPW = _B // _NW


def _sc_gather_max(x_flat, flat1, flat2):
    mesh = plsc.VectorSubcoreMesh(core_axis_name="c", subcore_axis_name="s")

    @functools.partial(
        pl.kernel,
        mesh=mesh,
        out_type=jax.ShapeDtypeStruct((_B,), jnp.float32),
        scratch_types=[
            pltpu.VMEM((2 * _BPW,), jnp.int32),
            pltpu.VMEM((2 * _BPW,), jnp.float32),
            pltpu.VMEM((_BPW,), jnp.float32),
            pltpu.SemaphoreType.DMA,
        ],
    )
    def k(x_hbm, f1_hbm, f2_hbm, out_hbm, flat_v, gath_v, out_v, sem):
        wid = lax.axis_index("s") * 2 + lax.axis_index("c")
        base = wid * _BPW
        pltpu.sync_copy(f1_hbm.at[pl.ds(base, _BPW)], flat_v.at[pl.ds(0, _BPW)])
        pltpu.sync_copy(f2_hbm.at[pl.ds(base, _BPW)], flat_v.at[pl.ds(_BPW, _BPW)])
        pltpu.async_copy(x_hbm.at[flat_v], gath_v, sem).wait()
        for j in range(_BPW // _L):
            v1 = gath_v[pl.ds(j * _L, _L)]
            v2 = gath_v[pl.ds(_BPW + j * _L, _L)]
            out_v[pl.ds(j * _L, _L)] = jnp.maximum(v1, v2)
        pltpu.sync_copy(out_v, out_hbm.at[pl.ds(base, _BPW)])

    return k(x_flat, flat1, flat2)


def kernel(x):
    B, N = x.shape
    key = jax.random.key(42)
    k1, k2 = jax.random.split(key)
    idx1 = jax.random.randint(k1, (B,), 0, N).astype(jnp.int32)
    idx2 = jax.random.randint(k2, (B,), 0, N).astype(jnp.int32)
    rows = jnp.arange(B, dtype=jnp.int32)
    flat1 = rows * N + idx1
    flat2 = rows * N + idx2
    x_flat = x.reshape(B * N)
    return _sc_gather_max(x_flat, flat1, flat2)


# speedup vs baseline: 2.2324x; 2.2324x over previous
import functools

import jax
import jax.numpy as jnp
from jax import lax
from jax.experimental import pallas as pl
from jax.experimental.pallas import tpu as pltpu
from jax.experimental.pallas import tpu_sc as plsc

_B = 1024
_N = 100000
_L = 16
_NW = 32
_BPW = _B // _NW


def _sc_gather_max(x, idx1, idx2):
    mesh = plsc.VectorSubcoreMesh(core_axis_name="c", subcore_axis_name="s")

    @functools.partial(
        pl.kernel,
        mesh=mesh,
        out_type=jax.ShapeDtypeStruct((_B,), jnp.float32),
        scratch_types=[
            pltpu.VMEM((2 * _BPW,), jnp.int32),
            pltpu.VMEM((2 * _BPW, 8, 128), jnp.float32),
            pltpu.VMEM_SHARED((_NW * 2 * _BPW * 128,), jnp.float32),
            pltpu.VMEM((2 * _BPW,), jnp.int32),
            pltpu.VMEM((2 * _BPW,), jnp.float32),
            pltpu.VMEM((_BPW,), jnp.float32),
            pltpu.SemaphoreType.DMA,
            pltpu.SemaphoreType.DMA,
        ],
    )
    def k(x_hbm, i1_hbm, i2_hbm, out_hbm, idx_v, seg_v, shr_v,
          pos_v, gath_v, out_v, sem, sem2):
        wid = lax.axis_index("s") * 2 + lax.axis_index("c")
        base = wid * _BPW
        pltpu.sync_copy(i1_hbm.at[pl.ds(base, _BPW)], idx_v.at[pl.ds(0, _BPW)])
        pltpu.sync_copy(i2_hbm.at[pl.ds(base, _BPW)], idx_v.at[pl.ds(_BPW, _BPW)])
        copies = []
        for i in range(2 * _BPW):
            c = idx_v[pl.ds((i // _L) * _L, _L)][i % _L]
            c_al = pl.multiple_of((c >> 7) << 7, 128)
            row_al = pl.multiple_of(base + ((i % _BPW) & ~7), 8)
            copies.append(pltpu.make_async_copy(
                x_hbm.at[pl.ds(row_al, 8), pl.ds(c_al, 128)],
                seg_v.at[i],
                sem,
            ))
        for cp in copies:
            cp.start()
        for cp in copies:
            cp.wait()
        nrow = 2 * _BPW * 128
        rowcps = []
        for i in range(2 * _BPW):
            r8 = (i % _BPW) & 7
            rowcps.append(pltpu.make_async_copy(
                seg_v.at[i, r8],
                shr_v.at[pl.ds(wid * nrow + i * 128, 128)], sem2))
        for cp in rowcps:
            cp.start()
        for cp in rowcps:
            cp.wait()
        pos_iota = lax.iota(jnp.int32, _L)
        for j in range(2 * _BPW // _L):
            iv = idx_v[pl.ds(j * _L, _L)]
            pos_v[pl.ds(j * _L, _L)] = (
                wid * nrow + (pos_iota + j * _L) * 128 + (iv & 127))
        pltpu.async_copy(shr_v.at[pos_v], gath_v, sem).wait()
        for j in range(_BPW // _L):
            v1 = gath_v[pl.ds(j * _L, _L)]
            v2 = gath_v[pl.ds(_BPW + j * _L, _L)]
            out_v[pl.ds(j * _L, _L)] = jnp.maximum(v1, v2)
        pltpu.sync_copy(out_v, out_hbm.at[pl.ds(base, _BPW)])

    return k(x, idx1, idx2)


def kernel(x):
    B, N = x.shape
    key = jax.random.key(42)
    k1, k2 = jax.random.split(key)
    idx1 = jax.random.randint(k1, (B,), 0, N).astype(jnp.int32)
    idx2 = jax.random.randint(k2, (B,), 0, N).astype(jnp.int32)
    return _sc_gather_max(x, idx1, idx2)


# speedup vs baseline: 20.0518x; 8.9822x over previous
import functools

import jax
import jax.numpy as jnp
from jax import lax
from jax.experimental import pallas as pl
from jax.experimental.pallas import tpu as pltpu
from jax.experimental.pallas import tpu_sc as plsc

_B = 1024
_N = 100000
_L = 16
_NW = 32
_BPW = _B // _NW
_NEG = float("-inf")


def _sc_gather_max(xT, idx1, idx2):
    mesh = plsc.VectorSubcoreMesh(core_axis_name="c", subcore_axis_name="s")

    @functools.partial(
        pl.kernel,
        mesh=mesh,
        out_type=jax.ShapeDtypeStruct((_B,), jnp.float32),
        scratch_types=[
            pltpu.VMEM((2 * _BPW,), jnp.int32),
            pltpu.VMEM((2 * _BPW, 8, 128), jnp.float32),
            pltpu.VMEM((_BPW,), jnp.float32),
            pltpu.SemaphoreType.DMA,
        ],
    )
    def k(xT_hbm, i1_hbm, i2_hbm, out_hbm, idx_v, seg_v, out_v, sem):
        wid = lax.axis_index("s") * 2 + lax.axis_index("c")
        base = wid * _BPW
        pltpu.sync_copy(i1_hbm.at[pl.ds(base, _BPW)], idx_v.at[pl.ds(0, _BPW)])
        pltpu.sync_copy(i2_hbm.at[pl.ds(base, _BPW)], idx_v.at[pl.ds(_BPW, _BPW)])

        def cscalar(i):
            return idx_v[pl.ds((i // _L) * _L, _L)][i % _L]

        bblk = pl.multiple_of((wid >> 2) * 128, 128)
        copies = []
        for i in range(2 * _BPW):
            c = cscalar(i)
            c_al = pl.multiple_of((c >> 3) << 3, 8)
            copies.append(pltpu.make_async_copy(
                xT_hbm.at[pl.ds(c_al, 8), pl.ds(bblk, 128)],
                seg_v.at[i],
                sem,
            ))
        for cp in copies:
            cp.start()
        for cp in copies:
            cp.wait()
        colbase = (wid & 3) * _BPW
        pos_iota = lax.iota(jnp.int32, _L)
        for j in range(_BPW // _L):
            st = pl.multiple_of(colbase + j * _L, _L)
            acc = jnp.full((_L,), _NEG, jnp.float32)
            for t in range(_L):
                for i in (j * _L + t, _BPW + j * _L + t):
                    c = cscalar(i)
                    v = seg_v[i, c & 7, pl.ds(st, _L)]
                    acc = jnp.maximum(acc, jnp.where(pos_iota == t, v, _NEG))
            out_v[pl.ds(j * _L, _L)] = acc
        pltpu.sync_copy(out_v, out_hbm.at[pl.ds(base, _BPW)])

    return k(xT, idx1, idx2)


def kernel(x):
    B, N = x.shape
    key = jax.random.key(42)
    k1, k2 = jax.random.split(key)
    idx1 = jax.random.randint(k1, (B,), 0, N).astype(jnp.int32)
    idx2 = jax.random.randint(k2, (B,), 0, N).astype(jnp.int32)
    return _sc_gather_max(x.T, idx1, idx2)


# speedup vs baseline: 36.8233x; 1.8364x over previous
import functools

import jax
import jax.numpy as jnp
from jax import lax
from jax.experimental import pallas as pl
from jax.experimental.pallas import tpu as pltpu
from jax.experimental.pallas import tpu_sc as plsc

_B = 1024
_N = 100000
_L = 16
_NW = 32
_BPW = _B // _NW
_NEG = float("-inf")


def _sc_gather_max(xT, idx_all):
    mesh = plsc.VectorSubcoreMesh(core_axis_name="c", subcore_axis_name="s")

    @functools.partial(
        pl.kernel,
        mesh=mesh,
        out_type=jax.ShapeDtypeStruct((_B,), jnp.float32),
        scratch_types=[
            pltpu.VMEM((2 * _BPW,), jnp.int32),
            pltpu.VMEM((2 * _BPW, _B), jnp.float32),
            pltpu.VMEM((_BPW,), jnp.float32),
            pltpu.SemaphoreType.DMA,
        ],
    )
    def k(xT_hbm, idx_hbm, out_hbm, idx_v, gath_v, out_v, sem):
        wid = lax.axis_index("s") * 2 + lax.axis_index("c")
        base = wid * _BPW
        pltpu.sync_copy(idx_hbm.at[pl.ds(wid * 2 * _BPW, 2 * _BPW)], idx_v)
        pltpu.async_copy(xT_hbm.at[idx_v], gath_v, sem).wait()
        pos_iota = lax.iota(jnp.int32, _L)
        for j in range(_BPW // _L):
            acc = jnp.full((_L,), _NEG, jnp.float32)
            for t in range(_L):
                for i in (j * _L + t, _BPW + j * _L + t):
                    v = gath_v[i, pl.ds(base + j * _L, _L)]
                    acc = jnp.maximum(acc, jnp.where(pos_iota == t, v, _NEG))
            out_v[pl.ds(j * _L, _L)] = acc
        pltpu.sync_copy(out_v, out_hbm.at[pl.ds(base, _BPW)])

    return k(xT, idx_all)


def kernel(x):
    B, N = x.shape
    with jax.ensure_compile_time_eval():
        key = jax.random.key(42)
        k1, k2 = jax.random.split(key)
        idx1 = jax.random.randint(k1, (B,), 0, N).astype(jnp.int32)
        idx2 = jax.random.randint(k2, (B,), 0, N).astype(jnp.int32)
        idx_all = jnp.concatenate(
            [idx1.reshape(_NW, _BPW), idx2.reshape(_NW, _BPW)], axis=1
        ).reshape(-1)
    return _sc_gather_max(x.T, idx_all)


# speedup vs baseline: 37.2977x; 1.0129x over previous
import functools

import jax
import jax.numpy as jnp
from jax import lax
from jax.experimental import pallas as pl
from jax.experimental.pallas import tpu as pltpu
from jax.experimental.pallas import tpu_sc as plsc

_B = 1024
_N = 100000
_L = 16
_NW = 32
_BPW = _B // _NW
_NEG = float("-inf")


def _sc_gather_max(xT, idx_all):
    mesh = plsc.VectorSubcoreMesh(core_axis_name="c", subcore_axis_name="s")

    @functools.partial(
        pl.kernel,
        mesh=mesh,
        out_type=jax.ShapeDtypeStruct((_B,), jnp.float32),
        scratch_types=[
            pltpu.VMEM((2 * _BPW,), jnp.int32),
            pltpu.VMEM((2 * _BPW, _B), jnp.float32),
            pltpu.VMEM((_BPW,), jnp.float32),
            pltpu.SemaphoreType.DMA,
        ],
    )
    def k(xT_hbm, idx_hbm, out_hbm, idx_v, gath_v, out_v, sem):
        wid = lax.axis_index("s") * 2 + lax.axis_index("c")
        base = wid * _BPW
        pltpu.sync_copy(idx_hbm.at[pl.ds(wid * 2 * _BPW, 2 * _BPW)], idx_v)
        pltpu.async_copy(xT_hbm.at[idx_v], gath_v, sem).wait()
        pos_iota = lax.iota(jnp.int32, _L)
        neg = jnp.full((_L,), _NEG, jnp.float32)

        def body(t, accs):
            new = []
            for j, acc in enumerate(accs):
                m = pos_iota == t
                for i in (j * _L + t, _BPW + j * _L + t):
                    v = gath_v[i, pl.ds(base + j * _L, _L)]
                    acc = jnp.maximum(acc, jnp.where(m, v, neg))
                new.append(acc)
            return tuple(new)

        accs = lax.fori_loop(0, _L, body, (neg, neg))
        for j in range(_BPW // _L):
            out_v[pl.ds(j * _L, _L)] = accs[j]
        pltpu.sync_copy(out_v, out_hbm.at[pl.ds(base, _BPW)])

    return k(xT, idx_all)


def kernel(x):
    B, N = x.shape
    with jax.ensure_compile_time_eval():
        key = jax.random.key(42)
        k1, k2 = jax.random.split(key)
        idx1 = jax.random.randint(k1, (B,), 0, N).astype(jnp.int32)
        idx2 = jax.random.randint(k2, (B,), 0, N).astype(jnp.int32)
        idx_all = jnp.concatenate(
            [idx1.reshape(_NW, _BPW), idx2.reshape(_NW, _BPW)], axis=1
        ).reshape(-1)
    return _sc_gather_max(x.T, idx_all)


# speedup vs baseline: 43.2143x; 1.1586x over previous
import functools

import jax
import jax.numpy as jnp
from jax import lax
from jax.experimental import pallas as pl
from jax.experimental.pallas import tpu as pltpu
from jax.experimental.pallas import tpu_sc as plsc

_B = 1024
_N = 100000
_L = 16
_NW = 32
_BPW = _B // _NW


def _sc_gather_max(x_words, pos_all):
    mesh = plsc.VectorSubcoreMesh(core_axis_name="c", subcore_axis_name="s")

    @functools.partial(
        pl.kernel,
        mesh=mesh,
        out_type=jax.ShapeDtypeStruct((_B,), jnp.float32),
        scratch_types=[
            pltpu.VMEM((2 * _BPW,), jnp.int32),
            pltpu.VMEM((2 * _BPW,), jnp.float32),
            pltpu.VMEM((_BPW,), jnp.float32),
            pltpu.SemaphoreType.DMA,
        ],
    )
    def k(x_hbm, pos_hbm, out_hbm, pos_v, gath_v, out_v, sem):
        wid = lax.axis_index("s") * 2 + lax.axis_index("c")
        base = wid * _BPW
        pltpu.sync_copy(pos_hbm.at[pl.ds(wid * 2 * _BPW, 2 * _BPW)], pos_v)
        pltpu.async_copy(x_hbm.at[pos_v], gath_v, sem).wait()
        for j in range(_BPW // _L):
            v1 = gath_v[pl.ds(j * _L, _L)]
            v2 = gath_v[pl.ds(_BPW + j * _L, _L)]
            out_v[pl.ds(j * _L, _L)] = jnp.maximum(v1, v2)
        pltpu.sync_copy(out_v, out_hbm.at[pl.ds(base, _BPW)])

    return k(x_words, pos_all)


def kernel(x):
    B, N = x.shape
    with jax.ensure_compile_time_eval():
        key = jax.random.key(42)
        k1, k2 = jax.random.split(key)
        idx1 = jax.random.randint(k1, (B,), 0, N).astype(jnp.int32)
        idx2 = jax.random.randint(k2, (B,), 0, N).astype(jnp.int32)
        b = jnp.arange(B, dtype=jnp.int32)

        def flatpos(c):
            return (((c >> 3) * 8 + (b >> 7)) * 1024
                    + (c & 7) * 128 + (b & 127))

        p1, p2 = flatpos(idx1), flatpos(idx2)
        pos_all = jnp.concatenate(
            [p1.reshape(_NW, _BPW), p2.reshape(_NW, _BPW)], axis=1
        ).reshape(-1)
    x_words = (x.T.reshape(N // 8, 8, 8, 128)
               .transpose(0, 2, 1, 3).reshape(-1))
    return _sc_gather_max(x_words, pos_all)


# speedup vs baseline: 46.2447x; 1.0701x over previous
import functools

import jax
import jax.numpy as jnp
from jax import lax
from jax.experimental import pallas as pl
from jax.experimental.pallas import tpu as pltpu
from jax.experimental.pallas import tpu_sc as plsc

_B = 1024
_N = 100000
_L = 16
_NW = 16
_BPW = _B // _NW


def _sc_gather_max(x_words, pos_all):
    mesh = plsc.VectorSubcoreMesh(
        core_axis_name="c", subcore_axis_name="s", num_cores=1)

    @functools.partial(
        pl.kernel,
        mesh=mesh,
        out_type=jax.ShapeDtypeStruct((_B,), jnp.float32),
        scratch_types=[
            pltpu.VMEM((2 * _BPW,), jnp.int32),
            pltpu.VMEM((2 * _BPW,), jnp.float32),
            pltpu.VMEM((_BPW,), jnp.float32),
            pltpu.SemaphoreType.DMA,
        ],
    )
    def k(x_hbm, pos_hbm, out_hbm, pos_v, gath_v, out_v, sem):
        wid = lax.axis_index("s") + lax.axis_index("c") * _NW
        base = wid * _BPW
        pltpu.sync_copy(pos_hbm.at[pl.ds(wid * 2 * _BPW, 2 * _BPW)], pos_v)
        pltpu.async_copy(x_hbm.at[pos_v], gath_v, sem).wait()
        for j in range(_BPW // _L):
            v1 = gath_v[pl.ds(j * _L, _L)]
            v2 = gath_v[pl.ds(_BPW + j * _L, _L)]
            out_v[pl.ds(j * _L, _L)] = jnp.maximum(v1, v2)
        pltpu.sync_copy(out_v, out_hbm.at[pl.ds(base, _BPW)])

    return k(x_words, pos_all)


def kernel(x):
    B, N = x.shape
    with jax.ensure_compile_time_eval():
        key = jax.random.key(42)
        k1, k2 = jax.random.split(key)
        idx1 = jax.random.randint(k1, (B,), 0, N).astype(jnp.int32)
        idx2 = jax.random.randint(k2, (B,), 0, N).astype(jnp.int32)
        b = jnp.arange(B, dtype=jnp.int32)

        def flatpos(c):
            return (((c >> 3) * 8 + (b >> 7)) * 1024
                    + (c & 7) * 128 + (b & 127))

        p1, p2 = flatpos(idx1), flatpos(idx2)
        pos_all = jnp.concatenate(
            [p1.reshape(_NW, _BPW), p2.reshape(_NW, _BPW)], axis=1
        ).reshape(-1)
    x_words = (x.T.reshape(N // 8, 8, 8, 128)
               .transpose(0, 2, 1, 3).reshape(-1))
    return _sc_gather_max(x_words, pos_all)
